# baseline (speedup 1.0000x reference)
import functools

import jax
import jax.numpy as jnp
from jax import lax
from jax.experimental import pallas as pl
from jax.experimental.pallas import tpu as pltpu
from jax.experimental.pallas import tpu_sc as plsc

NUM_EXPERTS = 8
TOP_K = 2
HIDDEN = 1024
INTERMEDIATE = 1024
NUM_TOKENS = 2048

NUM_PAIRS = NUM_TOKENS * TOP_K
ROW_TILE = 128
N_TILES = (NUM_PAIRS + NUM_EXPERTS * (ROW_TILE - 1) + ROW_TILE - 1) // ROW_TILE
N_ROWS = N_TILES * ROW_TILE


def _router_body(x_ref, wr_ref, pos_ref, w_ref, te_ref):
    x = x_ref[...]
    logits = jnp.dot(x, wr_ref[...], preferred_element_type=jnp.float32)
    p = jax.nn.softmax(logits, axis=-1)
    e_iota = lax.broadcasted_iota(jnp.int32, logits.shape, 1)
    m1 = jnp.max(logits, axis=-1, keepdims=True)
    i1 = jnp.min(jnp.where(logits == m1, e_iota, NUM_EXPERTS), axis=-1, keepdims=True)
    logits2 = jnp.where(e_iota == i1, -jnp.inf, logits)
    m2 = jnp.max(logits2, axis=-1, keepdims=True)
    i2 = jnp.min(jnp.where(logits2 == m2, e_iota, NUM_EXPERTS), axis=-1, keepdims=True)
    w1 = jnp.sum(jnp.where(e_iota == i1, p, 0.0), axis=-1, keepdims=True)
    w2 = jnp.sum(jnp.where(e_iota == i2, p, 0.0), axis=-1, keepdims=True)
    s = w1 + w2

    oh1 = (e_iota == i1).astype(jnp.float32)
    oh2 = (e_iota == i2).astype(jnp.float32)
    cnt2 = oh1 + oh2
    incl = cnt2
    shift = 1
    while shift < NUM_TOKENS:
        shifted = jnp.concatenate(
            [jnp.zeros((shift, NUM_EXPERTS), jnp.float32), incl[: NUM_TOKENS - shift, :]],
            axis=0,
        )
        incl = incl + shifted
        shift *= 2
    excl = incl - cnt2
    counts = incl[NUM_TOKENS - 1 : NUM_TOKENS, :]
    padcnt = jnp.ceil(counts / ROW_TILE) * ROW_TILE
    ip = padcnt
    sh = 1
    while sh < NUM_EXPERTS:
        ip = ip + jnp.concatenate(
            [jnp.zeros((1, sh), jnp.float32), ip[:, : NUM_EXPERTS - sh]], axis=1
        )
        sh *= 2
    off_pad = ip - padcnt

    rank1 = jnp.sum(jnp.where(e_iota == i1, excl, 0.0), axis=-1, keepdims=True)
    rank2 = jnp.sum(jnp.where(e_iota == i2, excl, 0.0), axis=-1, keepdims=True)
    base1 = jnp.sum(jnp.where(e_iota == i1, off_pad, 0.0), axis=-1, keepdims=True)
    base2 = jnp.sum(jnp.where(e_iota == i2, off_pad, 0.0), axis=-1, keepdims=True)
    pos1 = base1 + rank1
    pos2 = base2 + rank2
    pos_ref[...] = jnp.concatenate([pos1, pos2], axis=1).astype(jnp.int32)
    w_ref[...] = jnp.concatenate([w1 / s, w2 / s], axis=1)

    row0 = lax.broadcasted_iota(jnp.int32, (N_TILES, NUM_EXPERTS), 0).astype(jnp.float32) * ROW_TILE
    started = (jnp.broadcast_to(off_pad, (N_TILES, NUM_EXPERTS)) <= row0).astype(jnp.float32)
    te_ref[...] = jnp.sum(started, axis=-1, keepdims=True).astype(jnp.int32) - 1


def _router_sort(x, wr):
    return pl.pallas_call(
        _router_body,
        in_specs=[
            pl.BlockSpec((NUM_TOKENS, HIDDEN), lambda: (0, 0)),
            pl.BlockSpec((HIDDEN, NUM_EXPERTS), lambda: (0, 0)),
        ],
        out_specs=[
            pl.BlockSpec((NUM_TOKENS, TOP_K), lambda: (0, 0)),
            pl.BlockSpec((NUM_TOKENS, TOP_K), lambda: (0, 0)),
            pl.BlockSpec((N_TILES, 1), lambda: (0, 0)),
        ],
        out_shape=[
            jax.ShapeDtypeStruct((NUM_TOKENS, TOP_K), jnp.int32),
            jax.ShapeDtypeStruct((NUM_TOKENS, TOP_K), jnp.float32),
            jax.ShapeDtypeStruct((N_TILES, 1), jnp.int32),
        ],
    )(x, wr)


def _sc_scatter(pos_flat, w_flat):
    mesh = plsc.VectorSubcoreMesh(core_axis_name="c", subcore_axis_name="s")

    @functools.partial(
        pl.kernel,
        mesh=mesh,
        out_type=[
            jax.ShapeDtypeStruct((N_ROWS,), jnp.int32),
            jax.ShapeDtypeStruct((N_ROWS,), jnp.float32),
        ],
        scratch_types=[
            pltpu.VMEM((NUM_PAIRS,), jnp.int32),
            pltpu.VMEM((NUM_PAIRS,), jnp.float32),
            pltpu.VMEM((N_ROWS,), jnp.int32),
            pltpu.VMEM((N_ROWS,), jnp.float32),
        ],
        compiler_params=pltpu.CompilerParams(needs_layout_passes=False),
    )
    def k(pos_hbm, w_hbm, src_hbm, ws_hbm, pos_v, w_v, src_v, ws_v):
        cid = lax.axis_index("c")
        sid = lax.axis_index("s")

        @pl.when((cid == 0) & (sid == 0))
        def _():
            pltpu.sync_copy(pos_hbm, pos_v)
            pltpu.sync_copy(w_hbm, w_v)
            zi = jnp.zeros((16,), jnp.int32)
            zf = jnp.zeros((16,), jnp.float32)

            def zbody(i, _):
                src_v[pl.ds(i * 16, 16)] = zi
                ws_v[pl.ds(i * 16, 16)] = zf
                return 0

            lax.fori_loop(0, N_ROWS // 16, zbody, 0)

            tok0 = lax.shift_right_logical(lax.iota(jnp.int32, 16), 1)

            def sbody(i, tok):
                idx = pos_v[pl.ds(i * 16, 16)]
                plsc.store_scatter(src_v, [idx], tok)
                plsc.store_scatter(ws_v, [idx], w_v[pl.ds(i * 16, 16)])
                return tok + 8

            lax.fori_loop(0, NUM_PAIRS // 16, sbody, tok0)
            pltpu.sync_copy(src_v, src_hbm)
            pltpu.sync_copy(ws_v, ws_hbm)

    return k(pos_flat, w_flat)


def _sc_gather(table, idx, n_out):
    info = plsc.get_sparse_core_info()
    nw = info.num_cores * info.num_subcores
    b_per_w = n_out // nw
    chunk = 32
    n_chunks = b_per_w // chunk
    assert b_per_w % chunk == 0 and n_out % nw == 0
    mesh = plsc.VectorSubcoreMesh(core_axis_name="c", subcore_axis_name="s")

    @functools.partial(
        pl.kernel,
        mesh=mesh,
        out_type=jax.ShapeDtypeStruct((n_out, HIDDEN), jnp.float32),
        scratch_types=[
            pltpu.VMEM((b_per_w,), jnp.int32),
            pltpu.VMEM((chunk, HIDDEN), jnp.float32),
            pltpu.VMEM((chunk, HIDDEN), jnp.float32),
            pltpu.SemaphoreType.DMA,
            pltpu.SemaphoreType.DMA,
            pltpu.SemaphoreType.DMA,
            pltpu.SemaphoreType.DMA,
        ],
    )
    def k(table_hbm, idx_hbm, out_hbm, idx_v, buf0, buf1, g0, g1, w0, w1):
        wid = lax.axis_index("s") * info.num_cores + lax.axis_index("c")
        base = wid * b_per_w
        bufs = (buf0, buf1)
        gsem = (g0, g1)
        wsem = (w0, w1)
        pltpu.sync_copy(idx_hbm.at[pl.ds(base, b_per_w)], idx_v)
        gathers = [None, None]
        writes = [None, None]
        for c in range(n_chunks):
            b = c % 2
            if writes[b] is not None:
                writes[b].wait()
            gathers[b] = pltpu.async_copy(
                table_hbm.at[idx_v.at[pl.ds(c * chunk, chunk)]], bufs[b], gsem[b])
            if c > 0:
                pb = (c - 1) % 2
                gathers[pb].wait()
                writes[pb] = pltpu.async_copy(
                    bufs[pb], out_hbm.at[pl.ds(base + (c - 1) * chunk, chunk)], wsem[pb])
        lb = (n_chunks - 1) % 2
        gathers[lb].wait()
        writes[lb] = pltpu.async_copy(
            bufs[lb], out_hbm.at[pl.ds(base + (n_chunks - 1) * chunk, chunk)], wsem[lb])
        writes[(n_chunks - 2) % 2].wait()
        writes[lb].wait()

    return k(table, idx)


def _grouped_body(te_ref, src_ref, ws_ref, x_ref, wg_ref, wu_ref, wd_ref, y_ref):
    sv = src_ref[...]
    col = lax.broadcasted_iota(jnp.int32, (ROW_TILE, NUM_TOKENS), 1)
    perm = (col == sv).astype(jnp.float32)
    xt = jnp.dot(perm, x_ref[...], preferred_element_type=jnp.float32)
    g = jnp.dot(xt, wg_ref[0], preferred_element_type=jnp.float32)
    u = jnp.dot(xt, wu_ref[0], preferred_element_type=jnp.float32)
    h = (g * jax.nn.sigmoid(g)) * u * ws_ref[...]
    y_ref[...] = jnp.dot(h, wd_ref[0], preferred_element_type=jnp.float32)


def _grouped_mlp(x, src_col, ws_col, tile_expert, W_gate, W_up, W_down):
    grid_spec = pltpu.PrefetchScalarGridSpec(
        num_scalar_prefetch=1,
        grid=(N_TILES,),
        in_specs=[
            pl.BlockSpec((ROW_TILE, 1), lambda j, te: (j, 0)),
            pl.BlockSpec((ROW_TILE, 1), lambda j, te: (j, 0)),
            pl.BlockSpec((NUM_TOKENS, HIDDEN), lambda j, te: (0, 0)),
            pl.BlockSpec((1, HIDDEN, INTERMEDIATE), lambda j, te: (te[j], 0, 0)),
            pl.BlockSpec((1, HIDDEN, INTERMEDIATE), lambda j, te: (te[j], 0, 0)),
            pl.BlockSpec((1, INTERMEDIATE, HIDDEN), lambda j, te: (te[j], 0, 0)),
        ],
        out_specs=pl.BlockSpec((ROW_TILE, HIDDEN), lambda j, te: (j, 0)),
    )
    return pl.pallas_call(
        _grouped_body,
        grid_spec=grid_spec,
        out_shape=jax.ShapeDtypeStruct((N_ROWS, HIDDEN), jnp.float32),
        compiler_params=pltpu.CompilerParams(
            vmem_limit_bytes=100 * 1024 * 1024,
        ),
    )(tile_expert, src_col, ws_col, x, W_gate, W_up, W_down)


def _pair_sum_body(yp_ref, out_ref):
    yp = yp_ref[...]
    out_ref[...] = yp[:, :HIDDEN] + yp[:, HIDDEN:]


def _pair_sum(yp2):
    tile = 512
    return pl.pallas_call(
        _pair_sum_body,
        grid=(NUM_TOKENS // tile,),
        in_specs=[pl.BlockSpec((tile, 2 * HIDDEN), lambda t: (t, 0))],
        out_specs=pl.BlockSpec((tile, HIDDEN), lambda t: (t, 0)),
        out_shape=jax.ShapeDtypeStruct((NUM_TOKENS, HIDDEN), jnp.float32),
    )(yp2)


def kernel(hidden_states, W_router, W_gate, W_up, W_down):
    pos_pairs, w_pairs, tile_expert = _router_sort(hidden_states, W_router)
    pos_flat = pos_pairs.reshape(NUM_PAIRS)
    src, ws = _sc_scatter(pos_flat, w_pairs.reshape(NUM_PAIRS))
    ys = _grouped_mlp(hidden_states, src.reshape(N_ROWS, 1), ws.reshape(N_ROWS, 1),
                      tile_expert.reshape(N_TILES), W_gate, W_up, W_down)
    yp = _sc_gather(ys, pos_flat, NUM_PAIRS)
    return _pair_sum(yp.reshape(NUM_TOKENS, 2 * HIDDEN))


# speedup vs baseline: 1.0538x; 1.0538x over previous
import functools

import jax
import jax.numpy as jnp
from jax import lax
from jax.experimental import pallas as pl
from jax.experimental.pallas import tpu as pltpu
from jax.experimental.pallas import tpu_sc as plsc

NUM_EXPERTS = 8
TOP_K = 2
HIDDEN = 1024
INTERMEDIATE = 1024
NUM_TOKENS = 2048

NUM_PAIRS = NUM_TOKENS * TOP_K
ROW_TILE = 256
N_TILES = (NUM_PAIRS + NUM_EXPERTS * (ROW_TILE - 1) + ROW_TILE - 1) // ROW_TILE
N_ROWS = N_TILES * ROW_TILE


def _router_body(x_ref, wr_ref, pos_ref, w_ref, te_ref):
    x = x_ref[...]
    logits = jnp.dot(x, wr_ref[...], preferred_element_type=jnp.float32)
    p = jax.nn.softmax(logits, axis=-1)
    e_iota = lax.broadcasted_iota(jnp.int32, logits.shape, 1)
    m1 = jnp.max(logits, axis=-1, keepdims=True)
    i1 = jnp.min(jnp.where(logits == m1, e_iota, NUM_EXPERTS), axis=-1, keepdims=True)
    logits2 = jnp.where(e_iota == i1, -jnp.inf, logits)
    m2 = jnp.max(logits2, axis=-1, keepdims=True)
    i2 = jnp.min(jnp.where(logits2 == m2, e_iota, NUM_EXPERTS), axis=-1, keepdims=True)
    w1 = jnp.sum(jnp.where(e_iota == i1, p, 0.0), axis=-1, keepdims=True)
    w2 = jnp.sum(jnp.where(e_iota == i2, p, 0.0), axis=-1, keepdims=True)
    s = w1 + w2

    oh1 = (e_iota == i1).astype(jnp.float32)
    oh2 = (e_iota == i2).astype(jnp.float32)
    cnt2 = oh1 + oh2
    incl = cnt2
    shift = 1
    while shift < NUM_TOKENS:
        shifted = jnp.concatenate(
            [jnp.zeros((shift, NUM_EXPERTS), jnp.float32), incl[: NUM_TOKENS - shift, :]],
            axis=0,
        )
        incl = incl + shifted
        shift *= 2
    excl = incl - cnt2
    counts = incl[NUM_TOKENS - 1 : NUM_TOKENS, :]
    padcnt = jnp.ceil(counts / ROW_TILE) * ROW_TILE
    ip = padcnt
    sh = 1
    while sh < NUM_EXPERTS:
        ip = ip + jnp.concatenate(
            [jnp.zeros((1, sh), jnp.float32), ip[:, : NUM_EXPERTS - sh]], axis=1
        )
        sh *= 2
    off_pad = ip - padcnt

    rank1 = jnp.sum(jnp.where(e_iota == i1, excl, 0.0), axis=-1, keepdims=True)
    rank2 = jnp.sum(jnp.where(e_iota == i2, excl, 0.0), axis=-1, keepdims=True)
    base1 = jnp.sum(jnp.where(e_iota == i1, off_pad, 0.0), axis=-1, keepdims=True)
    base2 = jnp.sum(jnp.where(e_iota == i2, off_pad, 0.0), axis=-1, keepdims=True)
    pos1 = base1 + rank1
    pos2 = base2 + rank2
    pos_ref[...] = jnp.concatenate([pos1, pos2], axis=1).astype(jnp.int32)
    w_ref[...] = jnp.concatenate([w1 / s, w2 / s], axis=1)

    row0 = lax.broadcasted_iota(jnp.int32, (N_TILES, NUM_EXPERTS), 0).astype(jnp.float32) * ROW_TILE
    started = (jnp.broadcast_to(off_pad, (N_TILES, NUM_EXPERTS)) <= row0).astype(jnp.float32)
    te = jnp.sum(started, axis=-1, keepdims=True).astype(jnp.int32) - 1
    n_act = (ip[:, NUM_EXPERTS - 1 :] / ROW_TILE).astype(jnp.int32)
    te_ref[...] = jnp.concatenate([te, n_act], axis=0)


def _router_sort(x, wr):
    return pl.pallas_call(
        _router_body,
        in_specs=[
            pl.BlockSpec((NUM_TOKENS, HIDDEN), lambda: (0, 0)),
            pl.BlockSpec((HIDDEN, NUM_EXPERTS), lambda: (0, 0)),
        ],
        out_specs=[
            pl.BlockSpec((NUM_TOKENS, TOP_K), lambda: (0, 0)),
            pl.BlockSpec((NUM_TOKENS, TOP_K), lambda: (0, 0)),
            pl.BlockSpec((N_TILES + 1, 1), lambda: (0, 0)),
        ],
        out_shape=[
            jax.ShapeDtypeStruct((NUM_TOKENS, TOP_K), jnp.int32),
            jax.ShapeDtypeStruct((NUM_TOKENS, TOP_K), jnp.float32),
            jax.ShapeDtypeStruct((N_TILES + 1, 1), jnp.int32),
        ],
    )(x, wr)


def _sc_scatter(pos_flat, w_flat):
    mesh = plsc.VectorSubcoreMesh(core_axis_name="c", subcore_axis_name="s")

    @functools.partial(
        pl.kernel,
        mesh=mesh,
        out_type=[
            jax.ShapeDtypeStruct((N_ROWS,), jnp.int32),
            jax.ShapeDtypeStruct((N_ROWS,), jnp.float32),
        ],
        scratch_types=[
            pltpu.VMEM((NUM_PAIRS,), jnp.int32),
            pltpu.VMEM((NUM_PAIRS,), jnp.float32),
            pltpu.VMEM((N_ROWS,), jnp.int32),
            pltpu.VMEM((N_ROWS,), jnp.float32),
        ],
        compiler_params=pltpu.CompilerParams(needs_layout_passes=False),
    )
    def k(pos_hbm, w_hbm, src_hbm, ws_hbm, pos_v, w_v, src_v, ws_v):
        cid = lax.axis_index("c")
        sid = lax.axis_index("s")

        @pl.when((cid == 0) & (sid == 0))
        def _():
            pltpu.sync_copy(pos_hbm, pos_v)
            pltpu.sync_copy(w_hbm, w_v)
            zi = jnp.zeros((16,), jnp.int32)
            zf = jnp.zeros((16,), jnp.float32)

            def zbody(i, _):
                src_v[pl.ds(i * 16, 16)] = zi
                ws_v[pl.ds(i * 16, 16)] = zf
                return 0

            lax.fori_loop(0, N_ROWS // 16, zbody, 0)

            tok0 = lax.shift_right_logical(lax.iota(jnp.int32, 16), 1)

            def sbody(i, tok):
                idx = pos_v[pl.ds(i * 16, 16)]
                plsc.store_scatter(src_v, [idx], tok)
                plsc.store_scatter(ws_v, [idx], w_v[pl.ds(i * 16, 16)])
                return tok + 8

            lax.fori_loop(0, NUM_PAIRS // 16, sbody, tok0)
            pltpu.sync_copy(src_v, src_hbm)
            pltpu.sync_copy(ws_v, ws_hbm)

    return k(pos_flat, w_flat)


def _sc_gather(table, idx, n_out):
    info = plsc.get_sparse_core_info()
    nw = info.num_cores * info.num_subcores
    b_per_w = n_out // nw
    chunk = 32
    n_chunks = b_per_w // chunk
    assert b_per_w % chunk == 0 and n_out % nw == 0
    mesh = plsc.VectorSubcoreMesh(core_axis_name="c", subcore_axis_name="s")

    @functools.partial(
        pl.kernel,
        mesh=mesh,
        out_type=jax.ShapeDtypeStruct((n_out, HIDDEN), jnp.float32),
        scratch_types=[
            pltpu.VMEM((b_per_w,), jnp.int32),
            pltpu.VMEM((chunk, HIDDEN), jnp.float32),
            pltpu.VMEM((chunk, HIDDEN), jnp.float32),
            pltpu.SemaphoreType.DMA,
            pltpu.SemaphoreType.DMA,
            pltpu.SemaphoreType.DMA,
            pltpu.SemaphoreType.DMA,
        ],
    )
    def k(table_hbm, idx_hbm, out_hbm, idx_v, buf0, buf1, g0, g1, w0, w1):
        wid = lax.axis_index("s") * info.num_cores + lax.axis_index("c")
        base = wid * b_per_w
        bufs = (buf0, buf1)
        gsem = (g0, g1)
        wsem = (w0, w1)
        pltpu.sync_copy(idx_hbm.at[pl.ds(base, b_per_w)], idx_v)
        gathers = [None, None]
        writes = [None, None]
        for c in range(n_chunks):
            b = c % 2
            if writes[b] is not None:
                writes[b].wait()
            gathers[b] = pltpu.async_copy(
                table_hbm.at[idx_v.at[pl.ds(c * chunk, chunk)]], bufs[b], gsem[b])
            if c > 0:
                pb = (c - 1) % 2
                gathers[pb].wait()
                writes[pb] = pltpu.async_copy(
                    bufs[pb], out_hbm.at[pl.ds(base + (c - 1) * chunk, chunk)], wsem[pb])
        lb = (n_chunks - 1) % 2
        gathers[lb].wait()
        writes[lb] = pltpu.async_copy(
            bufs[lb], out_hbm.at[pl.ds(base + (n_chunks - 1) * chunk, chunk)], wsem[lb])
        writes[(n_chunks - 2) % 2].wait()
        writes[lb].wait()

    return k(table, idx)


def _grouped_body(te_ref, src_ref, ws_ref, x_ref, wg_ref, wu_ref, wd_ref, y_ref):
    j = pl.program_id(0)

    @pl.when(j < te_ref[N_TILES])
    def _():
        sv = src_ref[...]
        col = lax.broadcasted_iota(jnp.int32, (ROW_TILE, NUM_TOKENS), 1)
        perm = (col == sv).astype(jnp.float32)
        xt = jnp.dot(perm, x_ref[...], preferred_element_type=jnp.float32)
        g = jnp.dot(xt, wg_ref[0], preferred_element_type=jnp.float32)
        u = jnp.dot(xt, wu_ref[0], preferred_element_type=jnp.float32)
        h = (g * jax.nn.sigmoid(g)) * u * ws_ref[...]
        y_ref[...] = jnp.dot(h, wd_ref[0], preferred_element_type=jnp.float32)


def _grouped_mlp(x, src_col, ws_col, tile_expert, W_gate, W_up, W_down):
    grid_spec = pltpu.PrefetchScalarGridSpec(
        num_scalar_prefetch=1,
        grid=(N_TILES,),
        in_specs=[
            pl.BlockSpec((ROW_TILE, 1), lambda j, te: (j, 0)),
            pl.BlockSpec((ROW_TILE, 1), lambda j, te: (j, 0)),
            pl.BlockSpec((NUM_TOKENS, HIDDEN), lambda j, te: (0, 0)),
            pl.BlockSpec((1, HIDDEN, INTERMEDIATE), lambda j, te: (te[j], 0, 0)),
            pl.BlockSpec((1, HIDDEN, INTERMEDIATE), lambda j, te: (te[j], 0, 0)),
            pl.BlockSpec((1, INTERMEDIATE, HIDDEN), lambda j, te: (te[j], 0, 0)),
        ],
        out_specs=pl.BlockSpec((ROW_TILE, HIDDEN), lambda j, te: (j, 0)),
    )
    return pl.pallas_call(
        _grouped_body,
        grid_spec=grid_spec,
        out_shape=jax.ShapeDtypeStruct((N_ROWS, HIDDEN), jnp.float32),
        compiler_params=pltpu.CompilerParams(
            vmem_limit_bytes=100 * 1024 * 1024,
        ),
    )(tile_expert, src_col, ws_col, x, W_gate, W_up, W_down)


def _pair_sum_body(yp_ref, out_ref):
    yp = yp_ref[...]
    out_ref[...] = yp[:, :HIDDEN] + yp[:, HIDDEN:]


def _pair_sum(yp2):
    tile = 512
    return pl.pallas_call(
        _pair_sum_body,
        grid=(NUM_TOKENS // tile,),
        in_specs=[pl.BlockSpec((tile, 2 * HIDDEN), lambda t: (t, 0))],
        out_specs=pl.BlockSpec((tile, HIDDEN), lambda t: (t, 0)),
        out_shape=jax.ShapeDtypeStruct((NUM_TOKENS, HIDDEN), jnp.float32),
    )(yp2)


def kernel(hidden_states, W_router, W_gate, W_up, W_down):
    pos_pairs, w_pairs, tile_expert = _router_sort(hidden_states, W_router)
    pos_flat = pos_pairs.reshape(NUM_PAIRS)
    src, ws = _sc_scatter(pos_flat, w_pairs.reshape(NUM_PAIRS))
    ys = _grouped_mlp(hidden_states, src.reshape(N_ROWS, 1), ws.reshape(N_ROWS, 1),
                      tile_expert.reshape(N_TILES + 1), W_gate, W_up, W_down)
    yp = _sc_gather(ys, pos_flat, NUM_PAIRS)
    return _pair_sum(yp.reshape(NUM_TOKENS, 2 * HIDDEN))


# speedup vs baseline: 1.1677x; 1.1081x over previous
import functools

import jax
import jax.numpy as jnp
from jax import lax
from jax.experimental import pallas as pl
from jax.experimental.pallas import tpu as pltpu
from jax.experimental.pallas import tpu_sc as plsc

NUM_EXPERTS = 8
TOP_K = 2
HIDDEN = 1024
INTERMEDIATE = 1024
NUM_TOKENS = 2048

NUM_PAIRS = NUM_TOKENS * TOP_K
ROW_TILE = 256
N_TILES = (NUM_PAIRS + NUM_EXPERTS * (ROW_TILE - 1) + ROW_TILE - 1) // ROW_TILE
N_ROWS = N_TILES * ROW_TILE


def _router_body(x_ref, wr_ref, pos_ref, w_ref, te_ref):
    x = x_ref[...]
    logits = jnp.dot(x, wr_ref[...], preferred_element_type=jnp.float32)
    p = jax.nn.softmax(logits, axis=-1)
    e_iota = lax.broadcasted_iota(jnp.int32, logits.shape, 1)
    m1 = jnp.max(logits, axis=-1, keepdims=True)
    i1 = jnp.min(jnp.where(logits == m1, e_iota, NUM_EXPERTS), axis=-1, keepdims=True)
    logits2 = jnp.where(e_iota == i1, -jnp.inf, logits)
    m2 = jnp.max(logits2, axis=-1, keepdims=True)
    i2 = jnp.min(jnp.where(logits2 == m2, e_iota, NUM_EXPERTS), axis=-1, keepdims=True)
    w1 = jnp.sum(jnp.where(e_iota == i1, p, 0.0), axis=-1, keepdims=True)
    w2 = jnp.sum(jnp.where(e_iota == i2, p, 0.0), axis=-1, keepdims=True)
    s = w1 + w2

    oh1 = (e_iota == i1).astype(jnp.float32)
    oh2 = (e_iota == i2).astype(jnp.float32)
    cnt2 = oh1 + oh2
    incl = cnt2
    shift = 1
    while shift < NUM_TOKENS:
        shifted = jnp.concatenate(
            [jnp.zeros((shift, NUM_EXPERTS), jnp.float32), incl[: NUM_TOKENS - shift, :]],
            axis=0,
        )
        incl = incl + shifted
        shift *= 2
    excl = incl - cnt2
    counts = incl[NUM_TOKENS - 1 : NUM_TOKENS, :]
    padcnt = jnp.ceil(counts / ROW_TILE) * ROW_TILE
    ip = padcnt
    sh = 1
    while sh < NUM_EXPERTS:
        ip = ip + jnp.concatenate(
            [jnp.zeros((1, sh), jnp.float32), ip[:, : NUM_EXPERTS - sh]], axis=1
        )
        sh *= 2
    off_pad = ip - padcnt

    rank1 = jnp.sum(jnp.where(e_iota == i1, excl, 0.0), axis=-1, keepdims=True)
    rank2 = jnp.sum(jnp.where(e_iota == i2, excl, 0.0), axis=-1, keepdims=True)
    base1 = jnp.sum(jnp.where(e_iota == i1, off_pad, 0.0), axis=-1, keepdims=True)
    base2 = jnp.sum(jnp.where(e_iota == i2, off_pad, 0.0), axis=-1, keepdims=True)
    pos1 = base1 + rank1
    pos2 = base2 + rank2
    pos_ref[...] = jnp.concatenate([pos1, pos2], axis=1).astype(jnp.int32)
    w_ref[...] = jnp.concatenate([w1 / s, w2 / s], axis=1)

    row0 = lax.broadcasted_iota(jnp.int32, (N_TILES, NUM_EXPERTS), 0).astype(jnp.float32) * ROW_TILE
    started = (jnp.broadcast_to(off_pad, (N_TILES, NUM_EXPERTS)) <= row0).astype(jnp.float32)
    te = jnp.sum(started, axis=-1, keepdims=True).astype(jnp.int32) - 1
    n_act = (ip[:, NUM_EXPERTS - 1 :] / ROW_TILE).astype(jnp.int32)
    te_ref[...] = jnp.concatenate([te, n_act], axis=0)


def _router_sort(x, wr):
    return pl.pallas_call(
        _router_body,
        in_specs=[
            pl.BlockSpec((NUM_TOKENS, HIDDEN), lambda: (0, 0)),
            pl.BlockSpec((HIDDEN, NUM_EXPERTS), lambda: (0, 0)),
        ],
        out_specs=[
            pl.BlockSpec((NUM_TOKENS, TOP_K), lambda: (0, 0)),
            pl.BlockSpec((NUM_TOKENS, TOP_K), lambda: (0, 0)),
            pl.BlockSpec((N_TILES + 1, 1), lambda: (0, 0)),
        ],
        out_shape=[
            jax.ShapeDtypeStruct((NUM_TOKENS, TOP_K), jnp.int32),
            jax.ShapeDtypeStruct((NUM_TOKENS, TOP_K), jnp.float32),
            jax.ShapeDtypeStruct((N_TILES + 1, 1), jnp.int32),
        ],
    )(x, wr)


def _sc_scatter(pos_flat, w_flat):
    mesh = plsc.VectorSubcoreMesh(core_axis_name="c", subcore_axis_name="s")

    @functools.partial(
        pl.kernel,
        mesh=mesh,
        out_type=[
            jax.ShapeDtypeStruct((N_ROWS,), jnp.int32),
            jax.ShapeDtypeStruct((N_ROWS,), jnp.float32),
        ],
        scratch_types=[
            pltpu.VMEM((NUM_PAIRS,), jnp.int32),
            pltpu.VMEM((NUM_PAIRS,), jnp.float32),
            pltpu.VMEM((N_ROWS,), jnp.int32),
            pltpu.VMEM((N_ROWS,), jnp.float32),
        ],
        compiler_params=pltpu.CompilerParams(needs_layout_passes=False),
    )
    def k(pos_hbm, w_hbm, src_hbm, ws_hbm, pos_v, w_v, src_v, ws_v):
        cid = lax.axis_index("c")
        sid = lax.axis_index("s")

        @pl.when((cid == 0) & (sid == 0))
        def _():
            pltpu.sync_copy(pos_hbm, pos_v)
            pltpu.sync_copy(w_hbm, w_v)
            zi = jnp.zeros((16,), jnp.int32)
            zf = jnp.zeros((16,), jnp.float32)

            def zbody(i, _):
                src_v[pl.ds(i * 16, 16)] = zi
                ws_v[pl.ds(i * 16, 16)] = zf
                return 0

            lax.fori_loop(0, N_ROWS // 16, zbody, 0)

            tok0 = lax.shift_right_logical(lax.iota(jnp.int32, 16), 1)

            def sbody(i, tok):
                idx = pos_v[pl.ds(i * 16, 16)]
                plsc.store_scatter(src_v, [idx], tok)
                plsc.store_scatter(ws_v, [idx], w_v[pl.ds(i * 16, 16)])
                return tok + 8

            lax.fori_loop(0, NUM_PAIRS // 16, sbody, tok0)
            pltpu.sync_copy(src_v, src_hbm)
            pltpu.sync_copy(ws_v, ws_hbm)

    return k(pos_flat, w_flat)


def _sc_gather(table, idx, n_out):
    info = plsc.get_sparse_core_info()
    nw = info.num_cores * info.num_subcores
    b_per_w = n_out // nw
    chunk = 32
    n_chunks = b_per_w // chunk
    assert b_per_w % chunk == 0 and n_out % nw == 0
    mesh = plsc.VectorSubcoreMesh(core_axis_name="c", subcore_axis_name="s")

    @functools.partial(
        pl.kernel,
        mesh=mesh,
        out_type=jax.ShapeDtypeStruct((n_out, HIDDEN), jnp.float32),
        scratch_types=[
            pltpu.VMEM((b_per_w,), jnp.int32),
            pltpu.VMEM((chunk, HIDDEN), jnp.float32),
            pltpu.VMEM((chunk, HIDDEN), jnp.float32),
            pltpu.SemaphoreType.DMA,
            pltpu.SemaphoreType.DMA,
            pltpu.SemaphoreType.DMA,
            pltpu.SemaphoreType.DMA,
        ],
    )
    def k(table_hbm, idx_hbm, out_hbm, idx_v, buf0, buf1, g0, g1, w0, w1):
        wid = lax.axis_index("s") * info.num_cores + lax.axis_index("c")
        base = wid * b_per_w
        bufs = (buf0, buf1)
        gsem = (g0, g1)
        wsem = (w0, w1)
        pltpu.sync_copy(idx_hbm.at[pl.ds(base, b_per_w)], idx_v)
        gathers = [None, None]
        writes = [None, None]
        for c in range(n_chunks):
            b = c % 2
            if writes[b] is not None:
                writes[b].wait()
            gathers[b] = pltpu.async_copy(
                table_hbm.at[idx_v.at[pl.ds(c * chunk, chunk)]], bufs[b], gsem[b])
            if c > 0:
                pb = (c - 1) % 2
                gathers[pb].wait()
                writes[pb] = pltpu.async_copy(
                    bufs[pb], out_hbm.at[pl.ds(base + (c - 1) * chunk, chunk)], wsem[pb])
        lb = (n_chunks - 1) % 2
        gathers[lb].wait()
        writes[lb] = pltpu.async_copy(
            bufs[lb], out_hbm.at[pl.ds(base + (n_chunks - 1) * chunk, chunk)], wsem[lb])
        writes[(n_chunks - 2) % 2].wait()
        writes[lb].wait()

    return k(table, idx)


def _grouped_body(te_ref, src_ref, ws_ref, x_ref, wg_ref, wu_ref, wd_ref, y_ref):
    j = pl.program_id(0)

    @pl.when(j < te_ref[N_TILES])
    def _():
        sv = src_ref[...]
        col = lax.broadcasted_iota(jnp.int32, (ROW_TILE, NUM_TOKENS), 1)
        perm = (col == sv).astype(jnp.float32)
        xt = jnp.dot(perm, x_ref[...], preferred_element_type=jnp.float32)
        g = jnp.dot(xt, wg_ref[0], preferred_element_type=jnp.float32)
        u = jnp.dot(xt, wu_ref[0], preferred_element_type=jnp.float32)
        h = (g * jax.nn.sigmoid(g)) * u * ws_ref[...]
        y_ref[...] = jnp.dot(h, wd_ref[0], preferred_element_type=jnp.float32)


def _grouped_mlp(x, src_col, ws_col, tile_expert, W_gate, W_up, W_down):
    grid_spec = pltpu.PrefetchScalarGridSpec(
        num_scalar_prefetch=1,
        grid=(N_TILES,),
        in_specs=[
            pl.BlockSpec((ROW_TILE, 1), lambda j, te: (j, 0)),
            pl.BlockSpec((ROW_TILE, 1), lambda j, te: (j, 0)),
            pl.BlockSpec((NUM_TOKENS, HIDDEN), lambda j, te: (0, 0)),
            pl.BlockSpec((1, HIDDEN, INTERMEDIATE), lambda j, te: (te[j], 0, 0)),
            pl.BlockSpec((1, HIDDEN, INTERMEDIATE), lambda j, te: (te[j], 0, 0)),
            pl.BlockSpec((1, INTERMEDIATE, HIDDEN), lambda j, te: (te[j], 0, 0)),
        ],
        out_specs=pl.BlockSpec((ROW_TILE, HIDDEN), lambda j, te: (j, 0)),
    )
    return pl.pallas_call(
        _grouped_body,
        grid_spec=grid_spec,
        out_shape=jax.ShapeDtypeStruct((N_ROWS, HIDDEN), jnp.float32),
        compiler_params=pltpu.CompilerParams(
            vmem_limit_bytes=100 * 1024 * 1024,
        ),
    )(tile_expert, src_col, ws_col, x, W_gate, W_up, W_down)


def _pair_sum_body(yp_ref, out_ref):
    blk = yp_ref[...]
    n = blk.shape[0]
    row = lax.broadcasted_iota(jnp.int32, (n // 2, n), 0)
    col = lax.broadcasted_iota(jnp.int32, (n // 2, n), 1)
    s = (lax.shift_right_logical(col, 1) == row).astype(jnp.float32)
    out_ref[...] = jnp.dot(s, blk, preferred_element_type=jnp.float32)


def _pair_sum(yp):
    tile = 512
    return pl.pallas_call(
        _pair_sum_body,
        grid=(NUM_TOKENS // tile,),
        in_specs=[pl.BlockSpec((2 * tile, HIDDEN), lambda t: (t, 0))],
        out_specs=pl.BlockSpec((tile, HIDDEN), lambda t: (t, 0)),
        out_shape=jax.ShapeDtypeStruct((NUM_TOKENS, HIDDEN), jnp.float32),
    )(yp)


def kernel(hidden_states, W_router, W_gate, W_up, W_down):
    pos_pairs, w_pairs, tile_expert = _router_sort(hidden_states, W_router)
    pos_flat = pos_pairs.reshape(NUM_PAIRS)
    src, ws = _sc_scatter(pos_flat, w_pairs.reshape(NUM_PAIRS))
    ys = _grouped_mlp(hidden_states, src.reshape(N_ROWS, 1), ws.reshape(N_ROWS, 1),
                      tile_expert.reshape(N_TILES + 1), W_gate, W_up, W_down)
    yp = _sc_gather(ys, pos_flat, NUM_PAIRS)
    return _pair_sum(yp)


# speedup vs baseline: 1.2081x; 1.0346x over previous
import functools

import jax
import jax.numpy as jnp
from jax import lax
from jax.experimental import pallas as pl
from jax.experimental.pallas import tpu as pltpu
from jax.experimental.pallas import tpu_sc as plsc

NUM_EXPERTS = 8
TOP_K = 2
HIDDEN = 1024
INTERMEDIATE = 1024
NUM_TOKENS = 2048

NUM_PAIRS = NUM_TOKENS * TOP_K
ROW_TILE = 512
N_TILES = (NUM_PAIRS + NUM_EXPERTS * (ROW_TILE - 1) + ROW_TILE - 1) // ROW_TILE
N_ROWS = N_TILES * ROW_TILE


def _router_body(x_ref, wr_ref, pos_ref, w_ref, te_ref):
    x = x_ref[...]
    logits = jnp.dot(x, wr_ref[...], preferred_element_type=jnp.float32)
    p = jax.nn.softmax(logits, axis=-1)
    e_iota = lax.broadcasted_iota(jnp.int32, logits.shape, 1)
    m1 = jnp.max(logits, axis=-1, keepdims=True)
    i1 = jnp.min(jnp.where(logits == m1, e_iota, NUM_EXPERTS), axis=-1, keepdims=True)
    logits2 = jnp.where(e_iota == i1, -jnp.inf, logits)
    m2 = jnp.max(logits2, axis=-1, keepdims=True)
    i2 = jnp.min(jnp.where(logits2 == m2, e_iota, NUM_EXPERTS), axis=-1, keepdims=True)
    w1 = jnp.sum(jnp.where(e_iota == i1, p, 0.0), axis=-1, keepdims=True)
    w2 = jnp.sum(jnp.where(e_iota == i2, p, 0.0), axis=-1, keepdims=True)
    s = w1 + w2

    oh1 = (e_iota == i1).astype(jnp.float32)
    oh2 = (e_iota == i2).astype(jnp.float32)
    cnt2 = oh1 + oh2
    incl = cnt2
    shift = 1
    while shift < NUM_TOKENS:
        shifted = jnp.concatenate(
            [jnp.zeros((shift, NUM_EXPERTS), jnp.float32), incl[: NUM_TOKENS - shift, :]],
            axis=0,
        )
        incl = incl + shifted
        shift *= 2
    excl = incl - cnt2
    counts = incl[NUM_TOKENS - 1 : NUM_TOKENS, :]
    padcnt = jnp.ceil(counts / ROW_TILE) * ROW_TILE
    ip = padcnt
    sh = 1
    while sh < NUM_EXPERTS:
        ip = ip + jnp.concatenate(
            [jnp.zeros((1, sh), jnp.float32), ip[:, : NUM_EXPERTS - sh]], axis=1
        )
        sh *= 2
    off_pad = ip - padcnt

    rank1 = jnp.sum(jnp.where(e_iota == i1, excl, 0.0), axis=-1, keepdims=True)
    rank2 = jnp.sum(jnp.where(e_iota == i2, excl, 0.0), axis=-1, keepdims=True)
    base1 = jnp.sum(jnp.where(e_iota == i1, off_pad, 0.0), axis=-1, keepdims=True)
    base2 = jnp.sum(jnp.where(e_iota == i2, off_pad, 0.0), axis=-1, keepdims=True)
    pos1 = base1 + rank1
    pos2 = base2 + rank2
    pos_ref[...] = jnp.concatenate([pos1, pos2], axis=1).astype(jnp.int32)
    w_ref[...] = jnp.concatenate([w1 / s, w2 / s], axis=1)

    row0 = lax.broadcasted_iota(jnp.int32, (N_TILES, NUM_EXPERTS), 0).astype(jnp.float32) * ROW_TILE
    started = (jnp.broadcast_to(off_pad, (N_TILES, NUM_EXPERTS)) <= row0).astype(jnp.float32)
    te = jnp.sum(started, axis=-1, keepdims=True).astype(jnp.int32) - 1
    n_act = (ip[:, NUM_EXPERTS - 1 :] / ROW_TILE).astype(jnp.int32)
    te_ref[...] = jnp.concatenate([te, n_act], axis=0)


def _router_sort(x, wr):
    return pl.pallas_call(
        _router_body,
        in_specs=[
            pl.BlockSpec((NUM_TOKENS, HIDDEN), lambda: (0, 0)),
            pl.BlockSpec((HIDDEN, NUM_EXPERTS), lambda: (0, 0)),
        ],
        out_specs=[
            pl.BlockSpec((NUM_TOKENS, TOP_K), lambda: (0, 0)),
            pl.BlockSpec((NUM_TOKENS, TOP_K), lambda: (0, 0)),
            pl.BlockSpec((N_TILES + 1, 1), lambda: (0, 0)),
        ],
        out_shape=[
            jax.ShapeDtypeStruct((NUM_TOKENS, TOP_K), jnp.int32),
            jax.ShapeDtypeStruct((NUM_TOKENS, TOP_K), jnp.float32),
            jax.ShapeDtypeStruct((N_TILES + 1, 1), jnp.int32),
        ],
    )(x, wr)


def _sc_scatter(pos_flat, w_flat):
    mesh = plsc.VectorSubcoreMesh(core_axis_name="c", subcore_axis_name="s")

    @functools.partial(
        pl.kernel,
        mesh=mesh,
        out_type=[
            jax.ShapeDtypeStruct((N_ROWS,), jnp.int32),
            jax.ShapeDtypeStruct((N_ROWS,), jnp.float32),
        ],
        scratch_types=[
            pltpu.VMEM((NUM_PAIRS,), jnp.int32),
            pltpu.VMEM((NUM_PAIRS,), jnp.float32),
            pltpu.VMEM((N_ROWS,), jnp.int32),
            pltpu.VMEM((N_ROWS,), jnp.float32),
        ],
        compiler_params=pltpu.CompilerParams(needs_layout_passes=False),
    )
    def k(pos_hbm, w_hbm, src_hbm, ws_hbm, pos_v, w_v, src_v, ws_v):
        cid = lax.axis_index("c")
        sid = lax.axis_index("s")

        @pl.when((cid == 0) & (sid == 0))
        def _():
            pltpu.sync_copy(pos_hbm, pos_v)
            pltpu.sync_copy(w_hbm, w_v)
            zi = jnp.zeros((16,), jnp.int32)
            zf = jnp.zeros((16,), jnp.float32)

            def zbody(i, _):
                src_v[pl.ds(i * 16, 16)] = zi
                ws_v[pl.ds(i * 16, 16)] = zf
                return 0

            lax.fori_loop(0, N_ROWS // 16, zbody, 0)

            tok0 = lax.shift_right_logical(lax.iota(jnp.int32, 16), 1)

            def sbody(i, tok):
                idx = pos_v[pl.ds(i * 16, 16)]
                plsc.store_scatter(src_v, [idx], tok)
                plsc.store_scatter(ws_v, [idx], w_v[pl.ds(i * 16, 16)])
                return tok + 8

            lax.fori_loop(0, NUM_PAIRS // 16, sbody, tok0)
            pltpu.sync_copy(src_v, src_hbm)
            pltpu.sync_copy(ws_v, ws_hbm)

    return k(pos_flat, w_flat)


def _sc_gather(table, idx, n_out):
    info = plsc.get_sparse_core_info()
    nw = info.num_cores * info.num_subcores
    b_per_w = n_out // nw
    chunk = 32
    n_chunks = b_per_w // chunk
    assert b_per_w % chunk == 0 and n_out % nw == 0
    mesh = plsc.VectorSubcoreMesh(core_axis_name="c", subcore_axis_name="s")

    @functools.partial(
        pl.kernel,
        mesh=mesh,
        out_type=jax.ShapeDtypeStruct((n_out, HIDDEN), jnp.float32),
        scratch_types=[
            pltpu.VMEM((b_per_w,), jnp.int32),
            pltpu.VMEM((chunk, HIDDEN), jnp.float32),
            pltpu.VMEM((chunk, HIDDEN), jnp.float32),
            pltpu.SemaphoreType.DMA,
            pltpu.SemaphoreType.DMA,
            pltpu.SemaphoreType.DMA,
            pltpu.SemaphoreType.DMA,
        ],
    )
    def k(table_hbm, idx_hbm, out_hbm, idx_v, buf0, buf1, g0, g1, w0, w1):
        wid = lax.axis_index("s") * info.num_cores + lax.axis_index("c")
        base = wid * b_per_w
        bufs = (buf0, buf1)
        gsem = (g0, g1)
        wsem = (w0, w1)
        pltpu.sync_copy(idx_hbm.at[pl.ds(base, b_per_w)], idx_v)
        gathers = [None, None]
        writes = [None, None]
        for c in range(n_chunks):
            b = c % 2
            if writes[b] is not None:
                writes[b].wait()
            gathers[b] = pltpu.async_copy(
                table_hbm.at[idx_v.at[pl.ds(c * chunk, chunk)]], bufs[b], gsem[b])
            if c > 0:
                pb = (c - 1) % 2
                gathers[pb].wait()
                writes[pb] = pltpu.async_copy(
                    bufs[pb], out_hbm.at[pl.ds(base + (c - 1) * chunk, chunk)], wsem[pb])
        lb = (n_chunks - 1) % 2
        gathers[lb].wait()
        writes[lb] = pltpu.async_copy(
            bufs[lb], out_hbm.at[pl.ds(base + (n_chunks - 1) * chunk, chunk)], wsem[lb])
        writes[(n_chunks - 2) % 2].wait()
        writes[lb].wait()

    return k(table, idx)


def _grouped_body(te_ref, src_ref, ws_ref, x_ref, wg_ref, wu_ref, wd_ref, y_ref):
    j = pl.program_id(0)

    @pl.when(j < te_ref[N_TILES])
    def _():
        sv = src_ref[...]
        col = lax.broadcasted_iota(jnp.int32, (ROW_TILE, NUM_TOKENS), 1)
        perm = (col == sv).astype(jnp.float32)
        xt = jnp.dot(perm, x_ref[...], preferred_element_type=jnp.float32)
        g = jnp.dot(xt, wg_ref[0], preferred_element_type=jnp.float32)
        u = jnp.dot(xt, wu_ref[0], preferred_element_type=jnp.float32)
        h = (g * jax.nn.sigmoid(g)) * u * ws_ref[...]
        y_ref[...] = jnp.dot(h, wd_ref[0], preferred_element_type=jnp.float32)


def _grouped_mlp(x, src_col, ws_col, tile_expert, W_gate, W_up, W_down):
    grid_spec = pltpu.PrefetchScalarGridSpec(
        num_scalar_prefetch=1,
        grid=(N_TILES,),
        in_specs=[
            pl.BlockSpec((ROW_TILE, 1), lambda j, te: (j, 0)),
            pl.BlockSpec((ROW_TILE, 1), lambda j, te: (j, 0)),
            pl.BlockSpec((NUM_TOKENS, HIDDEN), lambda j, te: (0, 0)),
            pl.BlockSpec((1, HIDDEN, INTERMEDIATE), lambda j, te: (te[j], 0, 0)),
            pl.BlockSpec((1, HIDDEN, INTERMEDIATE), lambda j, te: (te[j], 0, 0)),
            pl.BlockSpec((1, INTERMEDIATE, HIDDEN), lambda j, te: (te[j], 0, 0)),
        ],
        out_specs=pl.BlockSpec((ROW_TILE, HIDDEN), lambda j, te: (j, 0)),
    )
    return pl.pallas_call(
        _grouped_body,
        grid_spec=grid_spec,
        out_shape=jax.ShapeDtypeStruct((N_ROWS, HIDDEN), jnp.float32),
        compiler_params=pltpu.CompilerParams(
            vmem_limit_bytes=100 * 1024 * 1024,
        ),
    )(tile_expert, src_col, ws_col, x, W_gate, W_up, W_down)


def _pair_sum_body(yp_ref, out_ref):
    blk = yp_ref[...]
    n = blk.shape[0]
    row = lax.broadcasted_iota(jnp.int32, (n // 2, n), 0)
    col = lax.broadcasted_iota(jnp.int32, (n // 2, n), 1)
    s = (lax.shift_right_logical(col, 1) == row).astype(jnp.float32)
    out_ref[...] = jnp.dot(s, blk, preferred_element_type=jnp.float32)


def _pair_sum(yp):
    tile = 512
    return pl.pallas_call(
        _pair_sum_body,
        grid=(NUM_TOKENS // tile,),
        in_specs=[pl.BlockSpec((2 * tile, HIDDEN), lambda t: (t, 0))],
        out_specs=pl.BlockSpec((tile, HIDDEN), lambda t: (t, 0)),
        out_shape=jax.ShapeDtypeStruct((NUM_TOKENS, HIDDEN), jnp.float32),
    )(yp)


def kernel(hidden_states, W_router, W_gate, W_up, W_down):
    pos_pairs, w_pairs, tile_expert = _router_sort(hidden_states, W_router)
    pos_flat = pos_pairs.reshape(NUM_PAIRS)
    src, ws = _sc_scatter(pos_flat, w_pairs.reshape(NUM_PAIRS))
    ys = _grouped_mlp(hidden_states, src.reshape(N_ROWS, 1), ws.reshape(N_ROWS, 1),
                      tile_expert.reshape(N_TILES + 1), W_gate, W_up, W_down)
    yp = _sc_gather(ys, pos_flat, NUM_PAIRS)
    return _pair_sum(yp)
